# baseline (speedup 1.0000x reference)
import jax
import jax.numpy as jnp
from jax import lax
from jax.experimental import pallas as pl
from jax.experimental.pallas import tpu as pltpu

_K = 1024
_D = 64
_N = 32768
_BLK = 512
_GRID = _N // _BLK
_BETA = 0.25


def _vq_body(x_ref, e_ref, et_ref, oh_ref, qst_ref, loss_ref, perp_ref,
             cnt_ref, acc_ref):
    i = pl.program_id(0)
    xc = x_ref[...].reshape(_D, _BLK)
    e = e_ref[...]
    et = et_ref[...]

    rn = jnp.sum(xc * xc, axis=0, keepdims=True)
    rnt = rn.T
    cn = jnp.sum(e * e, axis=1)
    mm = lax.dot_general(xc, et, (((0,), (0,)), ((), ())))
    dist = rnt + cn[None, :] - 2.0 * mm

    mn = jnp.min(dist, axis=1, keepdims=True)
    oh = (dist == mn).astype(jnp.float32)
    oh_ref[...] = oh
    cnt8 = jnp.dot(jnp.ones((8, _BLK), jnp.float32), oh)
    cnt0 = cnt8[0:1]
    nsel = jnp.sum(cnt0)

    @pl.when(i == 0)
    def _():
        cnt_ref[...] = jnp.zeros_like(cnt_ref)
        acc_ref[0] = 0.0

    @pl.when(nsel == jnp.float32(_BLK))
    def _():
        cnt_ref[...] += cnt0

    @pl.when(nsel != jnp.float32(_BLK))
    def _():
        iota = lax.broadcasted_iota(jnp.int32, (_BLK, _K), 1)
        idx = jnp.min(jnp.where(dist == mn, iota, _K), axis=1, keepdims=True)
        oh2 = (iota == idx).astype(jnp.float32)
        oh_ref[...] = oh2
        cnt_ref[...] += jnp.sum(oh2, axis=0, keepdims=True)

    ohf = oh_ref[...]
    qt = lax.dot_general(et, ohf, (((1,), (1,)), ((), ())))
    diff = qt - xc
    qst_ref[...] = (xc + diff).reshape(1, _D, _BLK)
    acc_ref[0] += jnp.sum(diff * diff)

    @pl.when(i == _GRID - 1)
    def _():
        m = acc_ref[0] / jnp.float32(_N * _D)
        loss_ref[...] = jnp.full((1, 1), m * _BETA + m, jnp.float32)
        avg = cnt_ref[...] / jnp.float32(_N)
        ent = jnp.sum(avg * jnp.log(avg + 1e-10))
        perp_ref[...] = jnp.full((1, 1), jnp.exp(-ent), jnp.float32)


def kernel(latents, embedding):
    b, c, h, w = latents.shape
    hw = h * w
    n_hw = hw // _BLK
    x3 = latents.reshape(b, c, hw)
    oh, qst, loss, perp = pl.pallas_call(
        _vq_body,
        grid=(_GRID,),
        in_specs=[
            pl.BlockSpec((1, _D, _BLK), lambda i: (i // n_hw, 0, i % n_hw)),
            pl.BlockSpec((_K, _D), lambda i: (0, 0)),
            pl.BlockSpec((_D, _K), lambda i: (0, 0)),
        ],
        out_specs=[
            pl.BlockSpec((_BLK, _K), lambda i: (i, 0)),
            pl.BlockSpec((1, _D, _BLK), lambda i: (i // n_hw, 0, i % n_hw)),
            pl.BlockSpec((1, 1), lambda i: (0, 0)),
            pl.BlockSpec((1, 1), lambda i: (0, 0)),
        ],
        out_shape=[
            jax.ShapeDtypeStruct((_N, _K), jnp.float32),
            jax.ShapeDtypeStruct((b, _D, hw), jnp.float32),
            jax.ShapeDtypeStruct((1, 1), jnp.float32),
            jax.ShapeDtypeStruct((1, 1), jnp.float32),
        ],
        scratch_shapes=[
            pltpu.VMEM((1, _K), jnp.float32),
            pltpu.SMEM((1,), jnp.float32),
        ],
        compiler_params=pltpu.CompilerParams(
            dimension_semantics=("arbitrary",)),
    )(x3, embedding, embedding.T)
    qst4 = qst.reshape(b, c, h, w)
    return (loss[0, 0], perp[0, 0], qst4, oh)


# speedup vs baseline: 1.2028x; 1.2028x over previous
import jax
import jax.numpy as jnp
from jax import lax
from jax.experimental import pallas as pl
from jax.experimental.pallas import tpu as pltpu

_K = 1024
_D = 64
_N = 32768
_BLK = 512
_GRID = _N // _BLK
_BETA = 0.25


def _vq_body(x_ref, e_ref, et2_ref, oh_ref, qst_ref, loss_ref, perp_ref,
             cnt_ref, acc_ref, cn_ref):
    i = pl.program_id(0)
    x = x_ref[...]
    e = e_ref[...]
    et2 = et2_ref[...]

    @pl.when(i == 0)
    def _():
        cnt_ref[...] = jnp.zeros_like(cnt_ref)
        acc_ref[0] = 0.0
        cn_ref[...] = jnp.sum(e * e, axis=1).reshape(1, _K)

    rn = jnp.sum(x * x, axis=1, keepdims=True)
    mm2 = jnp.dot(x, et2)
    dist = (rn + cn_ref[...]) - mm2

    mn = jnp.min(dist, axis=1, keepdims=True)
    oh = (dist == mn).astype(jnp.float32)
    oh_ref[...] = oh
    cnt8 = jnp.dot(jnp.ones((8, _BLK), jnp.float32), oh)
    cnt0 = cnt8[0:1]
    nsel = jnp.sum(cnt0)

    @pl.when(nsel == jnp.float32(_BLK))
    def _():
        cnt_ref[...] += cnt0

    @pl.when(nsel != jnp.float32(_BLK))
    def _():
        iota = lax.broadcasted_iota(jnp.int32, (_BLK, _K), 1)
        idx = jnp.min(jnp.where(dist == mn, iota, _K), axis=1, keepdims=True)
        oh2 = (iota == idx).astype(jnp.float32)
        oh_ref[...] = oh2
        cnt_ref[...] += jnp.sum(oh2, axis=0, keepdims=True)

    ohf = oh_ref[...]
    q = jnp.dot(ohf, e)
    diff = q - x
    qst_ref[...] = x + diff
    acc_ref[0] += jnp.sum(diff * diff)

    @pl.when(i == _GRID - 1)
    def _():
        m = acc_ref[0] / jnp.float32(_N * _D)
        loss_ref[...] = jnp.full((1, 1), m * _BETA + m, jnp.float32)
        avg = cnt_ref[...] / jnp.float32(_N)
        ent = jnp.sum(avg * jnp.log(avg + 1e-10))
        perp_ref[...] = jnp.full((1, 1), jnp.exp(-ent), jnp.float32)


def kernel(latents, embedding):
    b, c, h, w = latents.shape
    x = jnp.transpose(latents, (0, 2, 3, 1)).reshape(-1, _D)
    oh, qst, loss, perp = pl.pallas_call(
        _vq_body,
        grid=(_GRID,),
        in_specs=[
            pl.BlockSpec((_BLK, _D), lambda i: (i, 0)),
            pl.BlockSpec((_K, _D), lambda i: (0, 0)),
            pl.BlockSpec((_D, _K), lambda i: (0, 0)),
        ],
        out_specs=[
            pl.BlockSpec((_BLK, _K), lambda i: (i, 0)),
            pl.BlockSpec((_BLK, _D), lambda i: (i, 0)),
            pl.BlockSpec((1, 1), lambda i: (0, 0)),
            pl.BlockSpec((1, 1), lambda i: (0, 0)),
        ],
        out_shape=[
            jax.ShapeDtypeStruct((_N, _K), jnp.float32),
            jax.ShapeDtypeStruct((_N, _D), jnp.float32),
            jax.ShapeDtypeStruct((1, 1), jnp.float32),
            jax.ShapeDtypeStruct((1, 1), jnp.float32),
        ],
        scratch_shapes=[
            pltpu.VMEM((1, _K), jnp.float32),
            pltpu.SMEM((1,), jnp.float32),
            pltpu.VMEM((1, _K), jnp.float32),
        ],
        compiler_params=pltpu.CompilerParams(
            dimension_semantics=("arbitrary",)),
    )(x, embedding, embedding.T * 2.0)
    qst4 = jnp.transpose(qst.reshape(b, h, w, c), (0, 3, 1, 2))
    return (loss[0, 0], perp[0, 0], qst4, oh)


# speedup vs baseline: 1.3658x; 1.1355x over previous
import jax
import jax.numpy as jnp
from jax import lax
from jax.experimental import pallas as pl
from jax.experimental.pallas import tpu as pltpu

_K = 1024
_D = 64
_N = 32768
_BLK = 512
_GRID = _N // _BLK
_BETA = 0.25


def _vq_body(x_ref, e_ref, et2_ref, cn_ref, oh_ref, qst_ref, loss_ref,
             perp_ref, cnt_ref, acc_ref):
    i = pl.program_id(0)
    x = x_ref[...]
    e = e_ref[...]
    et2 = et2_ref[...]

    @pl.when(i == 0)
    def _():
        cnt_ref[...] = jnp.zeros_like(cnt_ref)
        acc_ref[0] = 0.0

    rn = jnp.sum(x * x, axis=1, keepdims=True)
    mm2 = jnp.dot(x, et2)
    dist = (rn + cn_ref[...]) - mm2

    mn = jnp.min(dist, axis=1, keepdims=True)
    oh = (dist == mn).astype(jnp.float32)
    oh_ref[...] = oh
    q = jnp.dot(oh, e)
    diff = q - x
    qst_ref[...] = x + diff
    acc_ref[0] += jnp.sum(diff * diff)
    cnt8 = jnp.dot(jnp.ones((8, _BLK), jnp.float32), oh)
    cnt0 = cnt8[0:1]
    cnt_ref[...] += cnt0
    nsel = jnp.sum(cnt0)

    @pl.when(nsel != jnp.float32(_BLK))
    def _():
        iota = lax.broadcasted_iota(jnp.int32, (_BLK, _K), 1)
        idx = jnp.min(jnp.where(dist == mn, iota, _K), axis=1, keepdims=True)
        oh2 = (iota == idx).astype(jnp.float32)
        oh_ref[...] = oh2
        q2 = jnp.dot(oh2, e)
        diff2 = q2 - x
        qst_ref[...] = x + diff2
        acc_ref[0] += jnp.sum(diff2 * diff2) - jnp.sum(diff * diff)
        cnt_ref[...] += jnp.sum(oh2, axis=0, keepdims=True) - cnt0

    @pl.when(i == _GRID - 1)
    def _():
        m = acc_ref[0] / jnp.float32(_N * _D)
        loss_ref[...] = jnp.full((1, 1), m * _BETA + m, jnp.float32)
        avg = cnt_ref[...] / jnp.float32(_N)
        ent = jnp.sum(avg * jnp.log(avg + 1e-10))
        perp_ref[...] = jnp.full((1, 1), jnp.exp(-ent), jnp.float32)


def kernel(latents, embedding):
    b, c, h, w = latents.shape
    x = jnp.transpose(latents, (0, 2, 3, 1)).reshape(-1, _D)
    cn = jnp.sum(embedding ** 2, axis=1)[None, :]
    oh, qst, loss, perp = pl.pallas_call(
        _vq_body,
        grid=(_GRID,),
        in_specs=[
            pl.BlockSpec((_BLK, _D), lambda i: (i, 0)),
            pl.BlockSpec((_K, _D), lambda i: (0, 0)),
            pl.BlockSpec((_D, _K), lambda i: (0, 0)),
            pl.BlockSpec((1, _K), lambda i: (0, 0)),
        ],
        out_specs=[
            pl.BlockSpec((_BLK, _K), lambda i: (i, 0)),
            pl.BlockSpec((_BLK, _D), lambda i: (i, 0)),
            pl.BlockSpec((1, 1), lambda i: (0, 0)),
            pl.BlockSpec((1, 1), lambda i: (0, 0)),
        ],
        out_shape=[
            jax.ShapeDtypeStruct((_N, _K), jnp.float32),
            jax.ShapeDtypeStruct((_N, _D), jnp.float32),
            jax.ShapeDtypeStruct((1, 1), jnp.float32),
            jax.ShapeDtypeStruct((1, 1), jnp.float32),
        ],
        scratch_shapes=[
            pltpu.VMEM((1, _K), jnp.float32),
            pltpu.SMEM((1,), jnp.float32),
        ],
        compiler_params=pltpu.CompilerParams(
            dimension_semantics=("arbitrary",)),
    )(x, embedding, embedding.T * 2.0, cn)
    qst4 = jnp.transpose(qst.reshape(b, h, w, c), (0, 3, 1, 2))
    return (loss[0, 0], perp[0, 0], qst4, oh)


# speedup vs baseline: 1.4636x; 1.0716x over previous
import jax
import jax.numpy as jnp
from jax import lax
from jax.experimental import pallas as pl
from jax.experimental.pallas import tpu as pltpu

_K = 1024
_D = 64
_N = 32768
_BLK = 1024
_GRID = _N // _BLK
_BETA = 0.25


def _vq_body(x_ref, e_ref, et2_ref, cn_ref, oh_ref, qst_ref, loss_ref,
             perp_ref, cnt_ref, acc_ref):
    i = pl.program_id(0)
    x = x_ref[...]
    e = e_ref[...]
    et2 = et2_ref[...]

    @pl.when(i == 0)
    def _():
        cnt_ref[...] = jnp.zeros_like(cnt_ref)
        acc_ref[0] = 0.0

    rn = jnp.sum(x * x, axis=1, keepdims=True)
    mm2 = jnp.dot(x, et2)
    dist = (rn + cn_ref[...]) - mm2

    mn = jnp.min(dist, axis=1, keepdims=True)
    oh = (dist == mn).astype(jnp.float32)
    oh_ref[...] = oh
    q = jnp.dot(oh, e)
    diff = q - x
    qst_ref[...] = x + diff
    acc_ref[0] += jnp.sum(diff * diff)
    cnt8 = jnp.dot(jnp.ones((8, _BLK), jnp.float32), oh)
    cnt0 = cnt8[0:1]
    cnt_ref[...] += cnt0
    nsel = jnp.sum(cnt0)

    @pl.when(nsel != jnp.float32(_BLK))
    def _():
        iota = lax.broadcasted_iota(jnp.int32, (_BLK, _K), 1)
        idx = jnp.min(jnp.where(dist == mn, iota, _K), axis=1, keepdims=True)
        oh2 = (iota == idx).astype(jnp.float32)
        oh_ref[...] = oh2
        q2 = jnp.dot(oh2, e)
        diff2 = q2 - x
        qst_ref[...] = x + diff2
        acc_ref[0] += jnp.sum(diff2 * diff2) - jnp.sum(diff * diff)
        cnt_ref[...] += jnp.sum(oh2, axis=0, keepdims=True) - cnt0

    @pl.when(i == _GRID - 1)
    def _():
        m = acc_ref[0] / jnp.float32(_N * _D)
        loss_ref[...] = jnp.full((1, 1), m * _BETA + m, jnp.float32)
        avg = cnt_ref[...] / jnp.float32(_N)
        ent = jnp.sum(avg * jnp.log(avg + 1e-10))
        perp_ref[...] = jnp.full((1, 1), jnp.exp(-ent), jnp.float32)


def kernel(latents, embedding):
    b, c, h, w = latents.shape
    x = jnp.transpose(latents, (0, 2, 3, 1)).reshape(-1, _D)
    cn = jnp.sum(embedding ** 2, axis=1)[None, :]
    oh, qst, loss, perp = pl.pallas_call(
        _vq_body,
        grid=(_GRID,),
        in_specs=[
            pl.BlockSpec((_BLK, _D), lambda i: (i, 0)),
            pl.BlockSpec((_K, _D), lambda i: (0, 0)),
            pl.BlockSpec((_D, _K), lambda i: (0, 0)),
            pl.BlockSpec((1, _K), lambda i: (0, 0)),
        ],
        out_specs=[
            pl.BlockSpec((_BLK, _K), lambda i: (i, 0)),
            pl.BlockSpec((_BLK, _D), lambda i: (i, 0)),
            pl.BlockSpec((1, 1), lambda i: (0, 0)),
            pl.BlockSpec((1, 1), lambda i: (0, 0)),
        ],
        out_shape=[
            jax.ShapeDtypeStruct((_N, _K), jnp.float32),
            jax.ShapeDtypeStruct((_N, _D), jnp.float32),
            jax.ShapeDtypeStruct((1, 1), jnp.float32),
            jax.ShapeDtypeStruct((1, 1), jnp.float32),
        ],
        scratch_shapes=[
            pltpu.VMEM((1, _K), jnp.float32),
            pltpu.SMEM((1,), jnp.float32),
        ],
        compiler_params=pltpu.CompilerParams(
            dimension_semantics=("arbitrary",)),
    )(x, embedding, embedding.T * 2.0, cn)
    qst4 = jnp.transpose(qst.reshape(b, h, w, c), (0, 3, 1, 2))
    return (loss[0, 0], perp[0, 0], qst4, oh)
